# baseline (speedup 1.0000x reference)
import jax
import jax.numpy as jnp
from jax.experimental import pallas as pl
from jax.experimental.pallas import tpu as pltpu


def _pool_kernel(x_ref, m_ref, o_ref):
    x = x_ref[...]
    m = m_ref[...][:, :, None]
    num = jnp.sum(x * m, axis=1)
    den = jnp.maximum(jnp.sum(m, axis=1), 1.0)
    o_ref[...] = (num * pl.reciprocal(den, approx=False)).astype(o_ref.dtype)


def kernel(features, input_mask):
    B, S, H = features.shape
    itemsize = jnp.dtype(features.dtype).itemsize
    mask_itemsize = jnp.dtype(input_mask.dtype).itemsize

    tb = 16 if B % 16 == 0 else (8 if B % 8 == 0 else B)
    grid = (pl.cdiv(B, tb),)

    feat_block = tb * S * H * itemsize
    mask_block = tb * S * mask_itemsize
    out_block = tb * H * itemsize
    vmem_limit = int(min(56 << 20, 2 * (feat_block + mask_block + out_block) + (12 << 20)))

    cost = pl.CostEstimate(
        flops=2 * B * S * H + B * S + B * H,
        transcendentals=0,
        bytes_accessed=B * S * H * itemsize + B * S * mask_itemsize + B * H * itemsize,
    )

    return pl.pallas_call(
        _pool_kernel,
        out_shape=jax.ShapeDtypeStruct((B, H), features.dtype),
        grid=grid,
        in_specs=[
            pl.BlockSpec((tb, S, H), lambda b: (b, 0, 0)),
            pl.BlockSpec((tb, S), lambda b: (b, 0)),
        ],
        out_specs=pl.BlockSpec((tb, H), lambda b: (b, 0)),
        compiler_params=pltpu.CompilerParams(
            dimension_semantics=("parallel",),
            vmem_limit_bytes=vmem_limit,
        ),
        cost_estimate=cost,
    )(features, input_mask)


# speedup vs baseline: 1.0167x; 1.0167x over previous
import jax
import jax.numpy as jnp
from jax.experimental import pallas as pl
from jax.experimental.pallas import tpu as pltpu


def _pool_kernel2(x0_ref, x1_ref, m_ref, o_ref):
    m = m_ref[...][:, :, None]
    hs = x0_ref.shape[1]
    num = jnp.sum(x0_ref[...] * m[:, :hs], axis=1) + jnp.sum(x1_ref[...] * m[:, hs:], axis=1)
    den = jnp.maximum(jnp.sum(m, axis=1), 1.0)
    o_ref[...] = (num * pl.reciprocal(den, approx=False)).astype(o_ref.dtype)


def _pool_kernel1(x_ref, m_ref, o_ref):
    m = m_ref[...][:, :, None]
    num = jnp.sum(x_ref[...] * m, axis=1)
    den = jnp.maximum(jnp.sum(m, axis=1), 1.0)
    o_ref[...] = (num * pl.reciprocal(den, approx=False)).astype(o_ref.dtype)


def kernel(features, input_mask):
    B, S, H = features.shape
    itemsize = jnp.dtype(features.dtype).itemsize
    mask_itemsize = jnp.dtype(input_mask.dtype).itemsize

    tb = 8 if B % 8 == 0 else B
    grid = (pl.cdiv(B, tb),)

    feat_block = tb * S * H * itemsize
    mask_block = tb * S * mask_itemsize
    out_block = tb * H * itemsize
    vmem_limit = int(min(56 << 20, 2 * (feat_block + mask_block + out_block) + (12 << 20)))

    cost = pl.CostEstimate(
        flops=2 * B * S * H + B * S + B * H,
        transcendentals=0,
        bytes_accessed=B * S * H * itemsize + B * S * mask_itemsize + B * H * itemsize,
    )
    compiler_params = pltpu.CompilerParams(
        dimension_semantics=("parallel",),
        vmem_limit_bytes=vmem_limit,
    )
    out_shape = jax.ShapeDtypeStruct((B, H), features.dtype)

    if (S // 2) % 8 != 0:
        return pl.pallas_call(
            _pool_kernel1,
            out_shape=out_shape,
            grid=grid,
            in_specs=[
                pl.BlockSpec((tb, S, H), lambda b: (b, 0, 0)),
                pl.BlockSpec((tb, S), lambda b: (b, 0)),
            ],
            out_specs=pl.BlockSpec((tb, H), lambda b: (b, 0)),
            compiler_params=compiler_params,
            cost_estimate=cost,
        )(features, input_mask)

    hs = S // 2
    return pl.pallas_call(
        _pool_kernel2,
        out_shape=out_shape,
        grid=grid,
        in_specs=[
            pl.BlockSpec((tb, hs, H), lambda b: (b, 0, 0)),
            pl.BlockSpec((tb, hs, H), lambda b: (b, 1, 0)),
            pl.BlockSpec((tb, S), lambda b: (b, 0)),
        ],
        out_specs=pl.BlockSpec((tb, H), lambda b: (b, 0)),
        compiler_params=compiler_params,
        cost_estimate=cost,
    )(features, features, input_mask)


# speedup vs baseline: 1.0230x; 1.0062x over previous
import jax
import jax.numpy as jnp
from jax.experimental import pallas as pl
from jax.experimental.pallas import tpu as pltpu


def _pool_kernel(x_ref, m_ref, o_ref):
    x = x_ref[...]
    m = m_ref[...][:, :, None]
    num = jnp.sum(x * m, axis=1)
    den = jnp.maximum(jnp.sum(m, axis=1), 1.0)
    o_ref[...] = (num * pl.reciprocal(den, approx=False)).astype(o_ref.dtype)


def kernel(features, input_mask):
    B, S, H = features.shape
    itemsize = jnp.dtype(features.dtype).itemsize
    mask_itemsize = jnp.dtype(input_mask.dtype).itemsize

    tb = 8 if B % 8 == 0 else B
    grid = (pl.cdiv(B, tb),)

    feat_block = tb * S * H * itemsize
    mask_block = tb * S * mask_itemsize
    out_block = tb * H * itemsize
    vmem_limit = int(min(56 << 20, 2 * (feat_block + mask_block + out_block) + (12 << 20)))

    cost = pl.CostEstimate(
        flops=2 * B * S * H + B * S + B * H,
        transcendentals=0,
        bytes_accessed=B * S * H * itemsize + B * S * mask_itemsize + B * H * itemsize,
    )

    return pl.pallas_call(
        _pool_kernel,
        out_shape=jax.ShapeDtypeStruct((B, H), features.dtype),
        grid=grid,
        in_specs=[
            pl.BlockSpec((tb, S, H), lambda b: (b, 0, 0)),
            pl.BlockSpec((tb, S), lambda b: (b, 0)),
        ],
        out_specs=pl.BlockSpec((tb, H), lambda b: (b, 0)),
        compiler_params=pltpu.CompilerParams(
            dimension_semantics=("parallel",),
            vmem_limit_bytes=vmem_limit,
        ),
        cost_estimate=cost,
    )(features, input_mask)
